# baseline (speedup 1.0000x reference)
import jax
import jax.numpy as jnp
from jax import lax
from jax.experimental import pallas as pl
from jax.experimental.pallas import tpu as pltpu
from jax.experimental.pallas import tpu_sc as plsc

RATIO = 0.5
CHUNK = 128
NPAD = 10240
TILE_N = NPAD // 16
NVREG = TILE_N // 16


def _score_body(x_ref, w_ref, b_ref, keys_ref, xs_ref):
    i = pl.program_id(0)
    s = jnp.dot(x_ref[...], w_ref[...]) + b_ref[0]
    xs_ref[...] = x_ref[...] * s
    s2d = s.reshape(16, 128)
    bits = lax.bitcast_convert_type(s2d, jnp.int32)
    key = jnp.where(s2d >= 0, jnp.bitwise_not(bits) & 0x7FFFFFFF, bits)
    row = (i * 2048 + lax.broadcasted_iota(jnp.int32, (16, 128), 0) * 128
           + lax.broadcasted_iota(jnp.int32, (16, 128), 1))
    keys_ref[...] = jnp.where(row < 10000, key, -1)


def _scores_keys_scaled(x, W, b):
    N, D = x.shape
    BLK = 2048
    keys2d, xs = pl.pallas_call(
        _score_body,
        grid=(NPAD // BLK,),
        in_specs=[
            pl.BlockSpec((BLK, D), lambda i: (i, 0)),
            pl.BlockSpec((D, 1), lambda i: (0, 0)),
            pl.BlockSpec(memory_space=pltpu.SMEM),
        ],
        out_specs=[
            pl.BlockSpec((16, 128), lambda i: (i, 0)),
            pl.BlockSpec((BLK, D), lambda i: (i, 0)),
        ],
        out_shape=[
            jax.ShapeDtypeStruct((NPAD // 128, 128), jnp.int32),
            jax.ShapeDtypeStruct((NPAD, D), jnp.float32),
        ],
    )(x, W.T, b)
    return keys2d.reshape(-1), xs


def _digits(k, shift):
    sh = lax.full(k.shape, shift, jnp.int32)
    return lax.shift_right_logical(k, sh) & 0xFF


def _sort_body(keys_hbm, out_hbm,
               ka, ia, kb, ib, hist_sp,
               kv, iv, locb, dstb, histv, basev, hv, sem):
    t = lax.axis_index("s")
    base = t * TILE_N

    def load_pair(src_k, src_i):
        pltpu.sync_copy(src_k.at[pl.ds(base, TILE_N)], kv)
        pltpu.sync_copy(src_i.at[pl.ds(base, TILE_N)], iv)

    def hist_and_loc(shift):
        for j in range(16):
            histv[pl.ds(j * 16, 16)] = jnp.zeros((16,), jnp.int32)

        for v in range(NVREG):
            k = kv[pl.ds(v * 16, 16)]
            dig = _digits(k, shift)
            cnt, last = plsc.scan_count(dig)
            prevh = plsc.load_gather(histv, [dig])
            locb[pl.ds(v * 16, 16)] = prevh + cnt - 1
            plsc.store_scatter(histv, [dig], prevh + cnt, mask=last)

        pltpu.sync_copy(histv, hist_sp.at[pl.ds(t * 256, 256)])
        plsc.subcore_barrier()

    def bucket_bases():
        pltpu.sync_copy(hist_sp, hv)
        carry = jnp.int32(0)
        zero = jnp.zeros((16,), jnp.int32)
        for j in range(16):
            tot, pre = zero, zero
            for r in range(16):
                row = hv[pl.ds(r * 256 + j * 16, 16)]
                pre = pre + jnp.where(r < t, row, zero)
                tot = tot + row
            ex = plsc.cumsum(tot) - tot
            basev[pl.ds(j * 16, 16)] = ex + carry + pre
            carry = carry + jnp.sum(tot)

    def scatter(shift, dst_k, dst_i, last_pass):
        handles = []
        for c in range(5):
            for u in range(8):
                v = c * 8 + u
                k = kv[pl.ds(v * 16, 16)]
                dig = _digits(k, shift)
                bse = plsc.load_gather(basev, [dig])
                dstb[c, pl.ds(u * 16, 16)] = bse + locb[pl.ds(v * 16, 16)]
            if not last_pass:
                handles.append(pltpu.async_copy(
                    kv.at[pl.ds(c * 128, 128)], dst_k.at[dstb.at[c]], sem))
            handles.append(pltpu.async_copy(
                iv.at[pl.ds(c * 128, 128)], dst_i.at[dstb.at[c]], sem))
        for h in handles:
            h.wait()
        plsc.subcore_barrier()

    pltpu.sync_copy(keys_hbm.at[pl.ds(base, TILE_N)], kv)
    for v in range(NVREG):
        iv[pl.ds(v * 16, 16)] = (base + v * 16
                                 + lax.broadcasted_iota(jnp.int32, (16,), 0))

    plan = [(0, None, ka, ia), (8, (ka, ia), kb, ib),
            (16, (kb, ib), ka, ia), (24, (ka, ia), ib, None)]
    for pi, (shift, src, dk, di) in enumerate(plan):
        if src is not None:
            with jax.named_scope(f"p{pi}_load"):
                load_pair(*src)
        with jax.named_scope(f"p{pi}_hist"):
            hist_and_loc(shift)
        with jax.named_scope(f"p{pi}_bases"):
            bucket_bases()
        with jax.named_scope(f"p{pi}_scat"):
            if di is None:
                scatter(shift, None, dk, True)
            else:
                scatter(shift, dk, di, False)
    with jax.named_scope("out_copy"):
        pltpu.sync_copy(ib.at[pl.ds(base, TILE_N)],
                        out_hbm.at[pl.ds(base, TILE_N)])


def _sc_sort(keys):
    mesh = plsc.VectorSubcoreMesh(core_axis_name="c", subcore_axis_name="s",
                                  num_cores=1)
    fn = pl.kernel(
        _sort_body,
        out_type=jax.ShapeDtypeStruct((NPAD,), jnp.int32),
        mesh=mesh,
        scratch_types=[
            pltpu.VMEM_SHARED((NPAD,), jnp.int32),
            pltpu.VMEM_SHARED((NPAD,), jnp.int32),
            pltpu.VMEM_SHARED((NPAD,), jnp.int32),
            pltpu.VMEM_SHARED((NPAD,), jnp.int32),
            pltpu.VMEM_SHARED((16 * 256,), jnp.int32),
            pltpu.VMEM((TILE_N,), jnp.int32),
            pltpu.VMEM((TILE_N,), jnp.int32),
            pltpu.VMEM((TILE_N,), jnp.int32),
            pltpu.VMEM((5, 128), jnp.int32),
            pltpu.VMEM((256,), jnp.int32),
            pltpu.VMEM((256,), jnp.int32),
            pltpu.VMEM((16 * 256,), jnp.int32),
            pltpu.SemaphoreType.DMA,
        ],
        compiler_params=pltpu.CompilerParams(needs_layout_passes=False),
    )
    return fn(keys)


def _gather_body(xs_hbm, idx_hbm, batch_hbm, out_hbm, bout_hbm,
                 idx_v, rows_v, bvals_v, idx_t, rows_t, bvals_t, bt_v, sem):
    w = lax.axis_index("s") * 2 + lax.axis_index("c")
    pltpu.sync_copy(batch_hbm, bt_v)

    def do_chunk(base, idx_b, rows_b, bvals_b, n):
        pltpu.sync_copy(idx_hbm.at[pl.ds(base, n)], idx_b)
        h = pltpu.async_copy(xs_hbm.at[idx_b], rows_b, sem)
        if n == CHUNK:
            for m in range(CHUNK // 16):
                idxv = idx_b[pl.ds(m * 16, 16)]
                bvals_b[pl.ds(m * 16, 16)] = plsc.load_gather(bt_v, [idxv])
            pltpu.sync_copy(bvals_b, bout_hbm.at[pl.ds(base, n)])
            h.wait()
        else:
            h.wait()
            pltpu.async_copy(batch_hbm.at[idx_b], bvals_b, sem).wait()
            pltpu.sync_copy(bvals_b, bout_hbm.at[pl.ds(base, n)])
        pltpu.sync_copy(rows_b, out_hbm.at[pl.ds(base, n)])

    do_chunk(w * CHUNK, idx_v, rows_v, bvals_v, CHUNK)

    @pl.when(w < 7)
    def _():
        do_chunk((w + 32) * CHUNK, idx_v, rows_v, bvals_v, CHUNK)

    @pl.when(w == 7)
    def _():
        do_chunk(39 * CHUNK, idx_t, rows_t, bvals_t, 8)


def _sc_gather(xs, top_idx, batch, k):
    D = xs.shape[1]
    mesh = plsc.VectorSubcoreMesh(core_axis_name="c", subcore_axis_name="s")
    fn = pl.kernel(
        _gather_body,
        out_type=[
            jax.ShapeDtypeStruct((k, D), jnp.float32),
            jax.ShapeDtypeStruct((k,), jnp.int32),
        ],
        mesh=mesh,
        scratch_types=[
            pltpu.VMEM((CHUNK,), jnp.int32),
            pltpu.VMEM((CHUNK, D), jnp.float32),
            pltpu.VMEM((CHUNK,), jnp.int32),
            pltpu.VMEM((8,), jnp.int32),
            pltpu.VMEM((8, D), jnp.float32),
            pltpu.VMEM((8,), jnp.int32),
            pltpu.VMEM((10000,), jnp.int32),
            pltpu.SemaphoreType.DMA,
        ],
        compiler_params=pltpu.CompilerParams(needs_layout_passes=False),
    )
    return fn(xs, top_idx, batch)


def kernel(x, edge_index, batch, W, b):
    N, D = x.shape
    k = max(1, int(N * RATIO))
    keys, xs = _scores_keys_scaled(x, W, b)
    top_idx = _sc_sort(keys)
    pooled_x, pooled_batch = _sc_gather(xs, top_idx, batch, k)
    return (pooled_x, edge_index, pooled_batch)


# speedup vs baseline: 1.0181x; 1.0181x over previous
import jax
import jax.numpy as jnp
from jax import lax
from jax.experimental import pallas as pl
from jax.experimental.pallas import tpu as pltpu
from jax.experimental.pallas import tpu_sc as plsc

RATIO = 0.5
CHUNK = 128
NPAD = 10240
TILE_N = NPAD // 16
NVREG = TILE_N // 16


def _score_body(x_ref, w_ref, b_ref, keys_ref, xs_ref):
    i = pl.program_id(0)
    s = jnp.dot(x_ref[...], w_ref[...]) + b_ref[0]
    xs_ref[...] = x_ref[...] * s
    s2d = s.reshape(16, 128)
    bits = lax.bitcast_convert_type(s2d, jnp.int32)
    key = jnp.where(s2d >= 0, jnp.bitwise_not(bits) & 0x7FFFFFFF, bits)
    row = (i * 2048 + lax.broadcasted_iota(jnp.int32, (16, 128), 0) * 128
           + lax.broadcasted_iota(jnp.int32, (16, 128), 1))
    keys_ref[...] = jnp.where(row < 10000, key, -1)


def _scores_keys_scaled(x, W, b):
    N, D = x.shape
    BLK = 2048
    keys2d, xs = pl.pallas_call(
        _score_body,
        grid=(NPAD // BLK,),
        in_specs=[
            pl.BlockSpec((BLK, D), lambda i: (i, 0)),
            pl.BlockSpec((D, 1), lambda i: (0, 0)),
            pl.BlockSpec(memory_space=pltpu.SMEM),
        ],
        out_specs=[
            pl.BlockSpec((16, 128), lambda i: (i, 0)),
            pl.BlockSpec((BLK, D), lambda i: (i, 0)),
        ],
        out_shape=[
            jax.ShapeDtypeStruct((NPAD // 128, 128), jnp.int32),
            jax.ShapeDtypeStruct((NPAD, D), jnp.float32),
        ],
    )(x, W.T, b)
    return keys2d.reshape(-1), xs


def _digits(k, shift):
    sh = lax.full(k.shape, shift, jnp.int32)
    return lax.shift_right_logical(k, sh) & 0xFF


def _sort_body(keys_hbm, out_hbm,
               ka, ia, kb, ib, hist_sp,
               kv, iv, locb, dstb, histv, basev, hv, sem):
    t = lax.axis_index("s")
    base = t * TILE_N

    def load_pair(src_k, src_i):
        pltpu.sync_copy(src_k.at[pl.ds(base, TILE_N)], kv)
        pltpu.sync_copy(src_i.at[pl.ds(base, TILE_N)], iv)

    def hist_and_loc(shift):
        for j in range(16):
            histv[pl.ds(j * 16, 16)] = jnp.zeros((16,), jnp.int32)

        for v in range(NVREG):
            k = kv[pl.ds(v * 16, 16)]
            dig = _digits(k, shift)
            cnt, last = plsc.scan_count(dig)
            prevh = plsc.load_gather(histv, [dig])
            locb[pl.ds(v * 16, 16)] = prevh + cnt - 1
            plsc.store_scatter(histv, [dig], prevh + cnt, mask=last)

        pltpu.sync_copy(histv, hist_sp.at[pl.ds(t * 256, 256)])
        plsc.subcore_barrier()

    def bucket_bases():
        pltpu.sync_copy(hist_sp, hv)
        carry = jnp.int32(0)
        zero = jnp.zeros((16,), jnp.int32)
        for j in range(16):
            tot, pre = zero, zero
            for r in range(16):
                row = hv[pl.ds(r * 256 + j * 16, 16)]
                pre = pre + jnp.where(r < t, row, zero)
                tot = tot + row
            ex = plsc.cumsum(tot) - tot
            basev[pl.ds(j * 16, 16)] = ex + carry + pre
            carry = carry + jnp.sum(tot)

    def scatter(shift, dst_k, dst_i, last_pass):
        handles = []
        for c in range(5):
            for u in range(8):
                v = c * 8 + u
                k = kv[pl.ds(v * 16, 16)]
                dig = _digits(k, shift)
                bse = plsc.load_gather(basev, [dig])
                dstb[c, pl.ds(u * 16, 16)] = bse + locb[pl.ds(v * 16, 16)]
            if not last_pass:
                handles.append(pltpu.async_copy(
                    kv.at[pl.ds(c * 128, 128)], dst_k.at[dstb.at[c]], sem))
            handles.append(pltpu.async_copy(
                iv.at[pl.ds(c * 128, 128)], dst_i.at[dstb.at[c]], sem))
        for h in handles:
            h.wait()
        plsc.subcore_barrier()

    pltpu.sync_copy(keys_hbm.at[pl.ds(base, TILE_N)], kv)
    for v in range(NVREG):
        iv[pl.ds(v * 16, 16)] = (base + v * 16
                                 + lax.broadcasted_iota(jnp.int32, (16,), 0))

    plan = [(0, None, ka, ia), (8, (ka, ia), kb, ib),
            (16, (kb, ib), ka, ia), (24, (ka, ia), ib, None)]
    for shift, src, dk, di in plan:
        if src is not None:
            load_pair(*src)
        hist_and_loc(shift)
        bucket_bases()
        if di is None:
            scatter(shift, None, dk, True)
        else:
            scatter(shift, dk, di, False)
    pltpu.sync_copy(ib.at[pl.ds(base, TILE_N)],
                    out_hbm.at[pl.ds(base, TILE_N)])


def _sc_sort(keys):
    mesh = plsc.VectorSubcoreMesh(core_axis_name="c", subcore_axis_name="s",
                                  num_cores=1)
    fn = pl.kernel(
        _sort_body,
        out_type=jax.ShapeDtypeStruct((NPAD,), jnp.int32),
        mesh=mesh,
        scratch_types=[
            pltpu.VMEM_SHARED((NPAD,), jnp.int32),
            pltpu.VMEM_SHARED((NPAD,), jnp.int32),
            pltpu.VMEM_SHARED((NPAD,), jnp.int32),
            pltpu.VMEM_SHARED((NPAD,), jnp.int32),
            pltpu.VMEM_SHARED((16 * 256,), jnp.int32),
            pltpu.VMEM((TILE_N,), jnp.int32),
            pltpu.VMEM((TILE_N,), jnp.int32),
            pltpu.VMEM((TILE_N,), jnp.int32),
            pltpu.VMEM((5, 128), jnp.int32),
            pltpu.VMEM((256,), jnp.int32),
            pltpu.VMEM((256,), jnp.int32),
            pltpu.VMEM((16 * 256,), jnp.int32),
            pltpu.SemaphoreType.DMA,
        ],
        compiler_params=pltpu.CompilerParams(needs_layout_passes=False),
    )
    return fn(keys)


def _gather_body(xs_hbm, idx_hbm, batch_hbm, out_hbm, bout_hbm,
                 idx_v, rows_v, bvals_v, idx_t, rows_t, bvals_t, sem):
    w = lax.axis_index("s") * 2 + lax.axis_index("c")

    def do_chunk(base, idx_b, rows_b, bvals_b, n):
        pltpu.sync_copy(idx_hbm.at[pl.ds(base, n)], idx_b)
        pltpu.async_copy(xs_hbm.at[idx_b], rows_b, sem).wait()
        pltpu.sync_copy(rows_b, out_hbm.at[pl.ds(base, n)])
        pltpu.async_copy(batch_hbm.at[idx_b], bvals_b, sem).wait()
        pltpu.sync_copy(bvals_b, bout_hbm.at[pl.ds(base, n)])

    do_chunk(w * CHUNK, idx_v, rows_v, bvals_v, CHUNK)

    @pl.when(w < 7)
    def _():
        do_chunk((w + 32) * CHUNK, idx_v, rows_v, bvals_v, CHUNK)

    @pl.when(w == 7)
    def _():
        do_chunk(39 * CHUNK, idx_t, rows_t, bvals_t, 8)


def _sc_gather(xs, top_idx, batch, k):
    D = xs.shape[1]
    mesh = plsc.VectorSubcoreMesh(core_axis_name="c", subcore_axis_name="s")
    fn = pl.kernel(
        _gather_body,
        out_type=[
            jax.ShapeDtypeStruct((k, D), jnp.float32),
            jax.ShapeDtypeStruct((k,), jnp.int32),
        ],
        mesh=mesh,
        scratch_types=[
            pltpu.VMEM((CHUNK,), jnp.int32),
            pltpu.VMEM((CHUNK, D), jnp.float32),
            pltpu.VMEM((CHUNK,), jnp.int32),
            pltpu.VMEM((8,), jnp.int32),
            pltpu.VMEM((8, D), jnp.float32),
            pltpu.VMEM((8,), jnp.int32),
            pltpu.SemaphoreType.DMA,
        ],
    )
    return fn(xs, top_idx, batch)


def kernel(x, edge_index, batch, W, b):
    N, D = x.shape
    k = max(1, int(N * RATIO))
    keys, xs = _scores_keys_scaled(x, W, b)
    top_idx = _sc_sort(keys)
    pooled_x, pooled_batch = _sc_gather(xs, top_idx, batch, k)
    return (pooled_x, edge_index, pooled_batch)
